# baseline (speedup 1.0000x reference)
import jax
import jax.numpy as jnp
from jax import lax
from jax.experimental import pallas as pl
from jax.experimental.pallas import tpu as pltpu, tpu_sc as plsc

_N = 50000
_NP = 50048
_L = 20
_D = 50
_F = _L * _D
_BS = 500
_NB = _N // _BS
_CH = 4096
_NFULL = _N // _CH
_TL0 = _NFULL * _CH
_TW = _NP - _TL0
_TREAL = _N - _TL0
_NCH = _NFULL + 1
_LC = 512
_NLFULL = _N // _LC
_LLAST = _N - _LC
_NC = 2
_NS = 16
_NW = _NC * _NS


def _chw(c):
    return (c * _CH, _CH) if c < _NFULL else (_TL0, _TW)


def _body(xT, idx, lengths, outT, outlen,
          idx_sh, stripA, stripB, cidx0, cidx1, outc0, outc1,
          lidx_v, lout_v, sem, psem, ssemA, ssemB, wsem0, wsem1):
    cidx = (cidx0, cidx1)
    outc = (outc0, outc1)
    wsem = (wsem0, wsem1)
    sid = lax.axis_index("s")
    wid = sid * _NC + lax.axis_index("c")

    @pl.when(sid == 0)
    def _():
        pltpu.sync_copy(idx, idx_sh)

    plsc.subcore_barrier()

    def len_body(i, carry):
        t = wid + i * _NW
        base = jnp.where(t < _NLFULL, t * _LC, _LLAST)
        pltpu.sync_copy(idx_sh.at[pl.ds(base, _LC)], lidx_v)
        pltpu.async_copy(lengths.at[lidx_v], lout_v, sem).wait()
        pltpu.sync_copy(lout_v, outlen.at[pl.ds(base, _LC)])
        return carry

    lax.fori_loop(0, (_NLFULL + 1 - wid + _NW - 1) // _NW, len_body, 0)

    ntrips = (_F - wid + _NW - 1) // _NW
    npairs = ntrips // 2
    iota = lax.iota(jnp.int32, 16)

    def drain(c, r):
        j0, w = _chw(c)
        pltpu.make_async_copy(
            outc[c % 2].at[pl.ds(0, w)],
            outT.at[r, pl.ds(j0, w)], wsem[c % 2]).wait()

    def gather_row(r, strip, has_prev):
        cp = pltpu.async_copy(idx_sh.at[pl.ds(0, _CH)], cidx[0], psem)
        for c in range(_NCH):
            j0, w = _chw(c)
            cp.wait()
            if c + 1 < _NCH:
                nj0, nw = _chw(c + 1)
                cp = pltpu.async_copy(idx_sh.at[pl.ds(nj0, nw)],
                                      cidx[(c + 1) % 2].at[pl.ds(0, nw)],
                                      psem)
            buf = cidx[c % 2]
            ob = outc[c % 2]
            if c >= 2:
                drain(c - 2, r)
            else:
                @pl.when(has_prev)
                def _():
                    drain(c + _NCH - 2, r)

            if c < _NFULL:
                @plsc.parallel_loop(0, _CH, step=16, unroll=16)
                def gather(j):
                    col = buf[pl.ds(j, 16)]
                    ob[pl.ds(j, 16)] = plsc.load_gather(strip, [col])
            else:
                @plsc.parallel_loop(0, _TW, step=16, unroll=8)
                def gather_tail(j):
                    col = buf[pl.ds(j, 16)]
                    m = (iota + j) < _TREAL
                    ob[pl.ds(j, 16)] = plsc.load_gather(strip, [col], mask=m)

            pltpu.async_copy(ob.at[pl.ds(0, w)],
                             outT.at[r, pl.ds(j0, w)],
                             wsem[c % 2])

    scpA = pltpu.async_copy(xT.at[wid], stripA, ssemA)

    def pair_body(i, carry):
        rA = wid + (2 * i) * _NW
        rB = rA + _NW
        pltpu.make_async_copy(xT.at[rA], stripA, ssemA).wait()
        pltpu.async_copy(xT.at[rB], stripB, ssemB)
        gather_row(rA, stripA, i > 0)

        pltpu.make_async_copy(xT.at[rB], stripB, ssemB).wait()
        rC = rB + _NW

        @pl.when(2 * i + 2 < ntrips)
        def _():
            pltpu.async_copy(xT.at[rC], stripA, ssemA)

        gather_row(rB, stripB, True)
        return carry

    lax.fori_loop(0, npairs, pair_body, 0)

    @pl.when(ntrips % 2 == 1)
    def _():
        r = wid + (ntrips - 1) * _NW
        pltpu.make_async_copy(xT.at[r], stripA, ssemA).wait()
        gather_row(r, stripA, npairs > 0)

    drain(_NCH - 2, wid)
    drain(_NCH - 1, wid)


def kernel(x, lengths, promptList, batchSize):
    xT = x.reshape(_N, _F).T
    idx = jnp.pad(promptList.astype(jnp.int32), (0, _NP - _N))
    mesh = plsc.VectorSubcoreMesh(core_axis_name="c", subcore_axis_name="s")
    outT, outlen = pl.kernel(
        _body,
        out_type=(
            jax.ShapeDtypeStruct((_F, _NP), jnp.float32),
            jax.ShapeDtypeStruct((_N,), jnp.int32),
        ),
        mesh=mesh,
        scratch_types=[
            pltpu.VMEM_SHARED((_NP,), jnp.int32),
            pltpu.VMEM((_N,), jnp.float32),
            pltpu.VMEM((_N,), jnp.float32),
            pltpu.VMEM((_CH,), jnp.int32),
            pltpu.VMEM((_CH,), jnp.int32),
            pltpu.VMEM((_CH,), jnp.float32),
            pltpu.VMEM((_CH,), jnp.float32),
            pltpu.VMEM((_LC,), jnp.int32),
            pltpu.VMEM((_LC,), jnp.int32),
            pltpu.SemaphoreType.DMA,
            pltpu.SemaphoreType.DMA,
            pltpu.SemaphoreType.DMA,
            pltpu.SemaphoreType.DMA,
            pltpu.SemaphoreType.DMA,
            pltpu.SemaphoreType.DMA,
        ],
        compiler_params=pltpu.CompilerParams(needs_layout_passes=False),
    )(xT, idx, lengths)
    xList = outT[:, :_N].reshape(_L, _D, _NB, _BS).transpose(2, 3, 0, 1)
    lengthList = outlen.reshape(_NB, _BS) + jnp.asarray(
        batchSize - _BS, dtype=outlen.dtype)
    return (xList, lengthList)


# speedup vs baseline: 2.9368x; 2.9368x over previous
import jax
import jax.numpy as jnp
from jax import lax
from jax.experimental import pallas as pl
from jax.experimental.pallas import tpu as pltpu, tpu_sc as plsc

_N = 50000
_L = 20
_D = 50
_F = _L * _D
_BS = 500
_BSP = 512
_NB = _N // _BS
_NBP = 104
_CB = 8
_NCH = _NBP // _CB
_CHJ = _CB * _BS
_CIW = 4096
_IP = (_NCH - 1) * _CHJ + _CIW
_LC = 512
_NLFULL = _N // _LC
_LLAST = _N - _LC
_NC = 2
_NS = 16
_NW = _NC * _NS


def _body(xP, idx, lengths, outT, outlen,
          idx_sh, stripA, stripB, cidx0, cidx1, outc0, outc1,
          lidx_v, lout_v, sem, psem, ssemA, ssemB, wsem0, wsem1):
    cidx = (cidx0, cidx1)
    outc = (outc0, outc1)
    wsem = (wsem0, wsem1)
    sid = lax.axis_index("s")
    wid = sid * _NC + lax.axis_index("c")

    @pl.when(sid == 0)
    def _():
        pltpu.sync_copy(idx, idx_sh)

    plsc.subcore_barrier()

    def len_body(i, carry):
        t = wid + i * _NW
        base = jnp.where(t < _NLFULL, t * _LC, _LLAST)
        pltpu.sync_copy(idx_sh.at[pl.ds(base, _LC)], lidx_v)
        pltpu.async_copy(lengths.at[lidx_v], lout_v, sem).wait()
        pltpu.sync_copy(lout_v, outlen.at[pl.ds(base, _LC)])
        return carry

    lax.fori_loop(0, (_NLFULL + 1 - wid + _NW - 1) // _NW, len_body, 0)

    ntrips = (_F - wid + _NW - 1) // _NW
    npairs = ntrips // 2

    def drain(c, r):
        pltpu.make_async_copy(
            outc[c % 2],
            outT.at[r, pl.ds(c * _CB, _CB), :], wsem[c % 2]).wait()

    def gather_row(r, strip, has_prev):
        cp = pltpu.async_copy(idx_sh.at[pl.ds(0, _CIW)], cidx[0], psem)
        for c in range(_NCH):
            cp.wait()
            if c + 1 < _NCH:
                cp = pltpu.async_copy(
                    idx_sh.at[pl.ds((c + 1) * _CHJ, _CIW)],
                    cidx[(c + 1) % 2], psem)
            buf = cidx[c % 2]
            ob = outc[c % 2]
            if c >= 2:
                drain(c - 2, r)
            else:
                @pl.when(has_prev)
                def _():
                    drain(c + _NCH - 2, r)

            @plsc.parallel_loop(0, _CB * _BSP, step=16, unroll=8)
            def gather(q):
                bi = q >> 9
                o = q & (_BSP - 1)
                col = buf[pl.ds(bi * _BS + o, 16)]
                ob[bi, pl.ds(o, 16)] = plsc.load_gather(strip, [col])

            pltpu.async_copy(ob, outT.at[r, pl.ds(c * _CB, _CB), :],
                             wsem[c % 2])

    def strip_src(r):
        return xP.at[r // _D, r % _D, :]

    scpA = pltpu.async_copy(strip_src(wid), stripA, ssemA)

    def pair_body(i, carry):
        rA = wid + (2 * i) * _NW
        rB = rA + _NW
        pltpu.make_async_copy(strip_src(rA), stripA, ssemA).wait()

        @pl.when(rB < _F)
        def _():
            pltpu.async_copy(strip_src(rB), stripB, ssemB)

        gather_row(rA, stripA, i > 0)

        @pl.when(rB < _F)
        def _():
            pltpu.make_async_copy(strip_src(rB), stripB, ssemB).wait()
            rC = rB + _NW

            @pl.when(rC < _F)
            def _():
                pltpu.async_copy(strip_src(rC), stripA, ssemA)

            gather_row(rB, stripB, True)

        return carry

    lax.fori_loop(0, (ntrips + 1) // 2, pair_body, 0)

    drain(_NCH - 2, wid)
    drain(_NCH - 1, wid)


def kernel(x, lengths, promptList, batchSize):
    xP = jnp.transpose(x, (1, 2, 0))
    idx = jnp.pad(promptList.astype(jnp.int32), (0, _IP - _N))
    mesh = plsc.VectorSubcoreMesh(core_axis_name="c", subcore_axis_name="s")
    outT, outlen = pl.kernel(
        _body,
        out_type=(
            jax.ShapeDtypeStruct((_F, _NBP, _BSP), jnp.float32),
            jax.ShapeDtypeStruct((_N,), jnp.int32),
        ),
        mesh=mesh,
        scratch_types=[
            pltpu.VMEM_SHARED((_IP,), jnp.int32),
            pltpu.VMEM((_N,), jnp.float32),
            pltpu.VMEM((_N,), jnp.float32),
            pltpu.VMEM((_CIW,), jnp.int32),
            pltpu.VMEM((_CIW,), jnp.int32),
            pltpu.VMEM((_CB, _BSP), jnp.float32),
            pltpu.VMEM((_CB, _BSP), jnp.float32),
            pltpu.VMEM((_LC,), jnp.int32),
            pltpu.VMEM((_LC,), jnp.int32),
            pltpu.SemaphoreType.DMA,
            pltpu.SemaphoreType.DMA,
            pltpu.SemaphoreType.DMA,
            pltpu.SemaphoreType.DMA,
            pltpu.SemaphoreType.DMA,
            pltpu.SemaphoreType.DMA,
        ],
        compiler_params=pltpu.CompilerParams(needs_layout_passes=False),
    )(xP, idx, lengths)
    xList = outT[:, :_NB, :_BS].reshape(_L, _D, _NB, _BS).transpose(2, 3, 0, 1)
    lengthList = outlen.reshape(_NB, _BS) + jnp.asarray(
        batchSize - _BS, dtype=outlen.dtype)
    return (xList, lengthList)


# speedup vs baseline: 2.9395x; 1.0009x over previous
import jax
import jax.numpy as jnp
from jax import lax
from jax.experimental import pallas as pl
from jax.experimental.pallas import tpu as pltpu, tpu_sc as plsc

_N = 50000
_L = 20
_D = 50
_F = _L * _D
_BS = 500
_BSP = 512
_NB = _N // _BS
_NBP = 104
_CB = 8
_NCH = _NBP // _CB
_CHJ = _CB * _BS
_CIW = 4096
_IP = (_NCH - 1) * _CHJ + _CIW
_LC = 512
_NLFULL = _N // _LC
_LLAST = _N - _LC
_NC = 2
_NS = 16
_NW = _NC * _NS


def _body(xP, idx, lengths, outT, outlen,
          idx_sh, stripA, stripB, cidx0, cidx1, outc0, outc1,
          lidx_v, lout_v, sem, psem, ssemA, ssemB, wsem0, wsem1):
    cidx = (cidx0, cidx1)
    outc = (outc0, outc1)
    wsem = (wsem0, wsem1)
    sid = lax.axis_index("s")
    wid = sid * _NC + lax.axis_index("c")

    @pl.when(sid == 0)
    def _():
        pltpu.sync_copy(idx, idx_sh)

    plsc.subcore_barrier()

    def len_body(i, carry):
        t = wid + i * _NW
        base = jnp.where(t < _NLFULL, t * _LC, _LLAST)
        pltpu.sync_copy(idx_sh.at[pl.ds(base, _LC)], lidx_v)
        pltpu.async_copy(lengths.at[lidx_v], lout_v, sem).wait()
        pltpu.sync_copy(lout_v, outlen.at[pl.ds(base, _LC)])
        return carry

    lax.fori_loop(0, (_NLFULL + 1 - wid + _NW - 1) // _NW, len_body, 0)

    ntrips = (_F - wid + _NW - 1) // _NW

    def drain(c, r):
        pltpu.make_async_copy(
            outc[c % 2],
            outT.at[r, pl.ds(c * _CB, _CB), :], wsem[c % 2]).wait()

    def gather_row(r, strip, has_prev):
        cp = pltpu.async_copy(idx_sh.at[pl.ds(0, _CIW)], cidx[0], psem)
        for c in range(_NCH):
            cp.wait()
            if c + 1 < _NCH:
                cp = pltpu.async_copy(
                    idx_sh.at[pl.ds((c + 1) * _CHJ, _CIW)],
                    cidx[(c + 1) % 2], psem)
            buf = cidx[c % 2]
            ob = outc[c % 2]
            if c >= 2:
                drain(c - 2, r)
            else:
                @pl.when(has_prev)
                def _():
                    drain(c + _NCH - 2, r)

            @plsc.parallel_loop(0, _CB * _BSP, step=16, unroll=8)
            def gather(q):
                bi = q >> 9
                o = q & (_BSP - 1)
                col = buf[pl.ds(bi * _BS + o, 16)]
                ob[bi, pl.ds(o, 16)] = plsc.load_gather(strip, [col])

            pltpu.async_copy(ob, outT.at[r, pl.ds(c * _CB, _CB), :],
                             wsem[c % 2])

    def strip_src(r):
        return xP.at[r // _D, r % _D, :]

    scpA = pltpu.async_copy(strip_src(wid), stripA, ssemA)

    def pair_body(i, carry):
        rA = wid + (2 * i) * _NW
        rB = rA + _NW
        pltpu.make_async_copy(strip_src(rA), stripA, ssemA).wait()

        @pl.when(rB < _F)
        def _():
            pltpu.async_copy(strip_src(rB), stripB, ssemB)

        gather_row(rA, stripA, i > 0)

        @pl.when(rB < _F)
        def _():
            pltpu.make_async_copy(strip_src(rB), stripB, ssemB).wait()
            rC = rB + _NW

            @pl.when(rC < _F)
            def _():
                pltpu.async_copy(strip_src(rC), stripA, ssemA)

            gather_row(rB, stripB, True)

        return carry

    lax.fori_loop(0, (ntrips + 1) // 2, pair_body, 0)

    drain(_NCH - 2, wid)
    drain(_NCH - 1, wid)


def kernel(x, lengths, promptList, batchSize):
    xP = jnp.transpose(x, (1, 2, 0))
    idx = jnp.pad(promptList.astype(jnp.int32), (0, _IP - _N))
    mesh = plsc.VectorSubcoreMesh(core_axis_name="c", subcore_axis_name="s")
    outT, outlen = pl.kernel(
        _body,
        out_type=(
            jax.ShapeDtypeStruct((_F, _NBP, _BSP), jnp.float32),
            jax.ShapeDtypeStruct((_N,), jnp.int32),
        ),
        mesh=mesh,
        scratch_types=[
            pltpu.VMEM_SHARED((_IP,), jnp.int32),
            pltpu.VMEM((_N,), jnp.float32),
            pltpu.VMEM((_N,), jnp.float32),
            pltpu.VMEM((_CIW,), jnp.int32),
            pltpu.VMEM((_CIW,), jnp.int32),
            pltpu.VMEM((_CB, _BSP), jnp.float32),
            pltpu.VMEM((_CB, _BSP), jnp.float32),
            pltpu.VMEM((_LC,), jnp.int32),
            pltpu.VMEM((_LC,), jnp.int32),
            pltpu.SemaphoreType.DMA,
            pltpu.SemaphoreType.DMA,
            pltpu.SemaphoreType.DMA,
            pltpu.SemaphoreType.DMA,
            pltpu.SemaphoreType.DMA,
            pltpu.SemaphoreType.DMA,
        ],
        compiler_params=pltpu.CompilerParams(needs_layout_passes=False),
    )(xP, idx, lengths)
    xList = outT[:, :_NB, :_BS].reshape(_L, _D, _NB, _BS).transpose(2, 3, 0, 1)
    lengthList = outlen.reshape(_NB, _BS) + jnp.asarray(
        batchSize - _BS, dtype=outlen.dtype)
    return (xList, lengthList)
